# baseline (speedup 1.0000x reference)
import jax
import jax.numpy as jnp
from jax.experimental import pallas as pl
from jax.experimental.pallas import tpu as pltpu

_N = 8388608
_ROWS = 65536
_CH = 8192
_NC = _ROWS // _CH
_NB = 4


def _body(t_ref, w_ref, o_ref, *scratch):
    bufs = scratch[:_NB]
    insems = scratch[_NB:2 * _NB]
    outsems = scratch[2 * _NB:3 * _NB]

    def in_cp(i):
        return pltpu.make_async_copy(
            w_ref.at[pl.ds(i * _CH, _CH), :], bufs[i % _NB], insems[i % _NB])

    def out_cp(i):
        return pltpu.make_async_copy(
            bufs[i % _NB], o_ref.at[pl.ds(i * _CH, _CH), :], outsems[i % _NB])

    for k in range(min(_NB, _NC)):
        in_cp(k).start()

    for i in range(_NC):
        in_cp(i).wait()
        if i == 0:
            buf = bufs[0]
            t = t_ref[0]
            tv = jnp.full((8, 128), t, dtype=jnp.float32)
            cond = (t > 500.0) & (t < 2502.54614894971)
            valv = 14.625 * jnp.where(
                cond, 0.01 * jnp.sin(0.001571 * (-500.0 + tv)), 0.0)
            ridx = jax.lax.broadcasted_iota(jnp.int32, (8, 128), 0)
            cidx = jax.lax.broadcasted_iota(jnp.int32, (8, 128), 1)
            first = (ridx == 0) & (cidx == 0)
            buf[0:8, :] = jnp.where(first, valv, buf[0:8, :])
        out_cp(i).start()
        nxt = i + 1
        if _NB <= nxt < _NC:
            out_cp(nxt - _NB).wait()
            in_cp(nxt).start()

    for i in range(max(_NC - _NB, 0), _NC):
        out_cp(i).wait()


def kernel(y, w, c, t):
    w2 = w.reshape(_ROWS, 128)
    t1 = t.reshape(1)
    out = pl.pallas_call(
        _body,
        in_specs=[
            pl.BlockSpec(memory_space=pltpu.SMEM),
            pl.BlockSpec(memory_space=pl.ANY),
        ],
        out_specs=pl.BlockSpec(memory_space=pl.ANY),
        out_shape=jax.ShapeDtypeStruct((_ROWS, 128), jnp.float32),
        scratch_shapes=[pltpu.VMEM((_CH, 128), jnp.float32)] * _NB
                       + [pltpu.SemaphoreType.DMA] * (2 * _NB),
    )(t1, w2)
    return out.reshape(_N)


# speedup vs baseline: 1.3283x; 1.3283x over previous
import jax
import jax.numpy as jnp
from jax.experimental import pallas as pl
from jax.experimental.pallas import tpu as pltpu

_N = 8388608
_ROWS = 65536
_GRID = 3
_BLOCK_ROWS = 24576


def _body(t_ref, w_ref, o_ref):
    o_ref[...] = w_ref[...]

    @pl.when(pl.program_id(0) == 0)
    def _():
        t = t_ref[0]
        tv = jnp.full((8, 128), t, dtype=jnp.float32)
        cond = (t > 500.0) & (t < 2502.54614894971)
        valv = 14.625 * jnp.where(cond, 0.01 * jnp.sin(0.001571 * (-500.0 + tv)), 0.0)
        ridx = jax.lax.broadcasted_iota(jnp.int32, (8, 128), 0)
        cidx = jax.lax.broadcasted_iota(jnp.int32, (8, 128), 1)
        first = (ridx == 0) & (cidx == 0)
        o_ref[0:8, :] = jnp.where(first, valv, w_ref[0:8, :])


def kernel(y, w, c, t):
    w2 = w.reshape(_ROWS, 128)
    t1 = t.reshape(1)
    out = pl.pallas_call(
        _body,
        grid=(_GRID,),
        in_specs=[
            pl.BlockSpec(memory_space=pltpu.SMEM),
            pl.BlockSpec((_BLOCK_ROWS, 128), lambda i: (i, 0)),
        ],
        out_specs=pl.BlockSpec((_BLOCK_ROWS, 128), lambda i: (i, 0)),
        out_shape=jax.ShapeDtypeStruct((_ROWS, 128), jnp.float32),
    )(t1, w2)
    return out.reshape(_N)


# speedup vs baseline: 1.3442x; 1.0120x over previous
import jax
import jax.numpy as jnp
from jax.experimental import pallas as pl
from jax.experimental.pallas import tpu as pltpu

_N = 8388608
_ROWS = 65536
_GRID = 3
_BLOCK_ROWS = 29952


def _body(t_ref, w_ref, o_ref):
    o_ref[...] = w_ref[...]

    @pl.when(pl.program_id(0) == 0)
    def _():
        t = t_ref[0]
        tv = jnp.full((8, 128), t, dtype=jnp.float32)
        cond = (t > 500.0) & (t < 2502.54614894971)
        valv = 14.625 * jnp.where(cond, 0.01 * jnp.sin(0.001571 * (-500.0 + tv)), 0.0)
        ridx = jax.lax.broadcasted_iota(jnp.int32, (8, 128), 0)
        cidx = jax.lax.broadcasted_iota(jnp.int32, (8, 128), 1)
        first = (ridx == 0) & (cidx == 0)
        o_ref[0:8, :] = jnp.where(first, valv, w_ref[0:8, :])


def kernel(y, w, c, t):
    w2 = w.reshape(_ROWS, 128)
    t1 = t.reshape(1)
    out = pl.pallas_call(
        _body,
        grid=(_GRID,),
        in_specs=[
            pl.BlockSpec(memory_space=pltpu.SMEM),
            pl.BlockSpec((_BLOCK_ROWS, 128), lambda i: (i, 0)),
        ],
        out_specs=pl.BlockSpec((_BLOCK_ROWS, 128), lambda i: (i, 0)),
        out_shape=jax.ShapeDtypeStruct((_ROWS, 128), jnp.float32),
    )(t1, w2)
    return out.reshape(_N)
